# Initial kernel scaffold; baseline (speedup 1.0000x reference)
import jax
import jax.numpy as jnp
from jax.experimental import pallas as pl


def kernel(input_feature, edge_index, edge_attr, W_e, b_e, eps, W1, b1, gamma1, beta1, mean1, var1, W2, b2, gamma2, beta2, mean2, var2):
    raise NotImplementedError("write your pallas kernel here")


# speedup vs baseline: 2.8292x; 2.8292x over previous
import functools
import jax
import jax.numpy as jnp
from jax import lax
from jax.experimental import pallas as pl
from jax.experimental.pallas import tpu as pltpu
from jax.experimental.pallas import tpu_sc as plsc

N = 10000
E = 320000
D = 128
DE = 16
DH = 256

NC = 2
NS = 16
NW = NC * NS
CHUNK = 128
NCHUNK = E // CHUNK
BASE_CH = NCHUNK // NW
EXTRA = NCHUNK - BASE_CH * NW
ROWS0 = 624
WB = 16


BE = 3200


def _emb_body(attr_ref, we_ref, be_ref, out_ref):
    out_ref[...] = (
        jnp.dot(attr_ref[...], we_ref[...], preferred_element_type=jnp.float32)
        + be_ref[...]
    )


def _edge_emb(edge_attr, W_e, b_e2):
    return pl.pallas_call(
        _emb_body,
        grid=(E // BE,),
        in_specs=[
            pl.BlockSpec((BE, DE), lambda i: (i, 0)),
            pl.BlockSpec((DE, D), lambda i: (0, 0)),
            pl.BlockSpec((1, D), lambda i: (0, 0)),
        ],
        out_specs=pl.BlockSpec((BE, D), lambda i: (i, 0)),
        out_shape=jax.ShapeDtypeStruct((E, D), jnp.float32),
    )(edge_attr, W_e, b_e2)


_sc_mesh = plsc.VectorSubcoreMesh(
    core_axis_name="core", subcore_axis_name="subcore",
    num_cores=NC, num_subcores=NS,
)


@functools.partial(
    pl.kernel,
    mesh=_sc_mesh,
    out_type=jax.ShapeDtypeStruct((NC, N, D), jnp.float32),
    scratch_types=[
        pltpu.VMEM((CHUNK,), jnp.int32),
        pltpu.VMEM((CHUNK,), jnp.int32),
        pltpu.VMEM((CHUNK, D), jnp.float32),
        pltpu.VMEM((CHUNK, D), jnp.float32),
        pltpu.VMEM((WB, D), jnp.float32),
        pltpu.VMEM_SHARED((N, D), jnp.float32),
    ],
)
def _sc_aggr(emb_hbm, src_hbm, dst_hbm, x_hbm, out_hbm,
             idx_s, idx_d, xbuf, ebuf, zbuf, aggr_sh):
    c = lax.axis_index("core")
    s = lax.axis_index("subcore")
    wid = c * NS + s

    @pl.loop(0, WB)
    def _z(r):
        for j in range(D // 16):
            zbuf.at[pl.ds(r, 1), pl.ds(j * 16, 16)][...] = jnp.zeros(
                (1, 16), jnp.float32)

    row0 = s * ROWS0

    @pl.loop(0, ROWS0 // WB)
    def _z0(k):
        pltpu.sync_copy(zbuf, aggr_sh.at[pl.ds(row0 + k * WB, WB)])

    @pl.when(s == NS - 1)
    def _z1():
        pltpu.sync_copy(zbuf, aggr_sh.at[pl.ds(NS * ROWS0, WB)])

    plsc.subcore_barrier()

    def process(ci):
        base = ci * CHUNK
        pltpu.sync_copy(src_hbm.at[pl.ds(base, CHUNK)], idx_s)
        pltpu.sync_copy(dst_hbm.at[pl.ds(base, CHUNK)], idx_d)
        pltpu.sync_copy(x_hbm.at[idx_s], xbuf)
        pltpu.sync_copy(emb_hbm.at[pl.ds(base, CHUNK)], ebuf)

        @pl.loop(0, CHUNK)
        def _c(r):
            for j in range(D // 16):
                sl = (pl.ds(r, 1), pl.ds(j * 16, 16))
                ebuf.at[sl][...] = jnp.maximum(
                    ebuf.at[sl][...] + xbuf.at[sl][...], 0.0)

        pltpu.sync_copy(ebuf, aggr_sh.at[idx_d], add=True)

    start = wid * BASE_CH + jnp.minimum(wid, EXTRA)

    @pl.loop(0, BASE_CH)
    def _main(i):
        process(start + i)

    @pl.when(wid < EXTRA)
    def _extra():
        process(wid * (BASE_CH + 1) + BASE_CH)

    plsc.subcore_barrier()

    def flush(r):
        pltpu.sync_copy(aggr_sh.at[pl.ds(r, WB)], zbuf)
        pltpu.sync_copy(zbuf, out_hbm.at[c, pl.ds(r, WB)])

    @pl.loop(0, ROWS0 // WB)
    def _w0(k):
        flush(row0 + k * WB)

    @pl.when(s == NS - 1)
    def _w1():
        flush(NS * ROWS0)


BN = 1000


def _mlp_body(x_ref, a0_ref, a1_ref, eps_ref, w1_ref, b1_ref, g1_ref, bt1_ref,
              m1_ref, v1_ref, w2_ref, b2_ref, g2_ref, bt2_ref, m2_ref, v2_ref,
              o_ref):
    h = x_ref[...] * (1.0 + eps_ref[0, 0]) + a0_ref[0] + a1_ref[0]
    t = jnp.dot(h, w1_ref[...], preferred_element_type=jnp.float32) + b1_ref[...]
    t = (t - m1_ref[...]) * lax.rsqrt(v1_ref[...] + 1e-5) * g1_ref[...] + bt1_ref[...]
    t = jnp.maximum(t, 0.0)
    o = jnp.dot(t, w2_ref[...], preferred_element_type=jnp.float32) + b2_ref[...]
    o_ref[...] = (o - m2_ref[...]) * lax.rsqrt(v2_ref[...] + 1e-5) * g2_ref[...] + bt2_ref[...]


def _mlp(x, aggr, eps2, W1, b1, g1, bt1, m1, v1, W2, b2, g2, bt2, m2, v2):
    row = lambda i: (i, 0)
    const2 = lambda i: (0, 0)
    return pl.pallas_call(
        _mlp_body,
        grid=(N // BN,),
        in_specs=[
            pl.BlockSpec((BN, D), row),
            pl.BlockSpec((1, BN, D), lambda i: (0, i, 0)),
            pl.BlockSpec((1, BN, D), lambda i: (1, i, 0)),
            pl.BlockSpec((1, 1), const2),
            pl.BlockSpec((D, DH), const2),
            pl.BlockSpec((1, DH), const2),
            pl.BlockSpec((1, DH), const2),
            pl.BlockSpec((1, DH), const2),
            pl.BlockSpec((1, DH), const2),
            pl.BlockSpec((1, DH), const2),
            pl.BlockSpec((DH, D), const2),
            pl.BlockSpec((1, D), const2),
            pl.BlockSpec((1, D), const2),
            pl.BlockSpec((1, D), const2),
            pl.BlockSpec((1, D), const2),
            pl.BlockSpec((1, D), const2),
        ],
        out_specs=pl.BlockSpec((BN, D), row),
        out_shape=jax.ShapeDtypeStruct((N, D), jnp.float32),
    )(x, aggr, aggr, eps2, W1, b1, g1, bt1, m1, v1, W2, b2, g2, bt2, m2, v2)


def kernel(input_feature, edge_index, edge_attr, W_e, b_e, eps, W1, b1,
           gamma1, beta1, mean1, var1, W2, b2, gamma2, beta2, mean2, var2):
    src = edge_index[0]
    dst = edge_index[1]
    emb = _edge_emb(edge_attr, W_e, b_e.reshape(1, D))
    aggr = _sc_aggr(emb, src, dst, input_feature)
    out = _mlp(
        input_feature, aggr, jnp.reshape(eps, (1, 1)),
        W1, b1.reshape(1, DH), gamma1.reshape(1, DH), beta1.reshape(1, DH),
        mean1.reshape(1, DH), var1.reshape(1, DH),
        W2, b2.reshape(1, D), gamma2.reshape(1, D), beta2.reshape(1, D),
        mean2.reshape(1, D), var2.reshape(1, D),
    )
    return out


# speedup vs baseline: 4.4236x; 1.5635x over previous
import functools
import jax
import jax.numpy as jnp
from jax import lax
from jax.experimental import pallas as pl
from jax.experimental.pallas import tpu as pltpu
from jax.experimental.pallas import tpu_sc as plsc

N = 10000
E = 320000
D = 128
DE = 16
DH = 256

NC = 2
NS = 16
NW = NC * NS
CHUNK = 64
NCHUNK = E // CHUNK
BASE_CH = NCHUNK // NW
EXTRA = NCHUNK - BASE_CH * NW
ROWS0 = 624
NSLOT = 6


BE = 3200


def _emb_body(attr_ref, we_ref, be_ref, out_ref):
    out_ref[...] = (
        jnp.dot(attr_ref[...], we_ref[...], preferred_element_type=jnp.float32)
        + be_ref[...]
    )


def _edge_emb(edge_attr, W_e, b_e2):
    return pl.pallas_call(
        _emb_body,
        grid=(E // BE,),
        in_specs=[
            pl.BlockSpec((BE, DE), lambda i: (i, 0)),
            pl.BlockSpec((DE, D), lambda i: (0, 0)),
            pl.BlockSpec((1, D), lambda i: (0, 0)),
        ],
        out_specs=pl.BlockSpec((BE, D), lambda i: (i, 0)),
        out_shape=jax.ShapeDtypeStruct((E, D), jnp.float32),
    )(edge_attr, W_e, b_e2)


_sc_mesh = plsc.VectorSubcoreMesh(
    core_axis_name="core", subcore_axis_name="subcore",
    num_cores=NC, num_subcores=NS,
)


@functools.partial(
    pl.kernel,
    mesh=_sc_mesh,
    out_type=jax.ShapeDtypeStruct((NC, N, D), jnp.float32),
    scratch_types=[
        pltpu.VMEM((NSLOT, 2, CHUNK), jnp.int32),
        pltpu.VMEM((CHUNK, D), jnp.float32),
        pltpu.VMEM((CHUNK, D), jnp.float32),
        pltpu.VMEM((CHUNK, D), jnp.float32),
        pltpu.VMEM((CHUNK, D), jnp.float32),
        pltpu.VMEM((CHUNK, D), jnp.float32),
        pltpu.VMEM_SHARED((N, D), jnp.float32),
        pltpu.SemaphoreType.DMA,
        pltpu.SemaphoreType.DMA,
        pltpu.SemaphoreType.DMA,
        pltpu.SemaphoreType.DMA,
        pltpu.SemaphoreType.DMA,
        pltpu.SemaphoreType.DMA,
        pltpu.SemaphoreType.DMA,
        pltpu.SemaphoreType.DMA,
        pltpu.SemaphoreType.DMA,
        pltpu.SemaphoreType.DMA,
    ],
)
def _sc_aggr(emb_hbm, src_hbm, dst_hbm, x_hbm, out_hbm,
             idx_v, xb0, xb1, eb0, eb1, eb2, aggr_sh,
             si0, si1, sx0, sx1, se0, se1, se2, ss0, ss1, ss2):
    c = lax.axis_index("core")
    s = lax.axis_index("subcore")
    wid = c * NS + s
    xb = (xb0, xb1)
    eb = (eb0, eb1, eb2)
    si = (si0, si1)
    sx = (sx0, sx1)
    se = (se0, se1, se2)
    ss = (ss0, ss1, ss2)

    @pl.loop(0, CHUNK)
    def _z(r):
        for j in range(D // 16):
            xb0.at[pl.ds(r, 1), pl.ds(j * 16, 16)][...] = jnp.zeros(
                (1, 16), jnp.float32)

    row0 = s * ROWS0
    for k in range(ROWS0 // CHUNK):
        pltpu.sync_copy(xb0, aggr_sh.at[pl.ds(row0 + k * CHUNK, CHUNK)])
    pltpu.sync_copy(xb0.at[pl.ds(0, 48)],
                    aggr_sh.at[pl.ds(row0 + 576, 48)])

    @pl.when(s == NS - 1)
    def _z1():
        pltpu.sync_copy(xb0.at[pl.ds(0, 16)], aggr_sh.at[pl.ds(NS * ROWS0, 16)])

    plsc.subcore_barrier()

    start = wid * BASE_CH + jnp.minimum(wid, EXTRA)

    def issue_idx(k, p):
        base = (start + k) * CHUNK
        slot = lax.rem(k, NSLOT)
        pltpu.async_copy(src_hbm.at[pl.ds(base, CHUNK)], idx_v.at[slot, 0],
                         si[p])
        pltpu.async_copy(dst_hbm.at[pl.ds(base, CHUNK)], idx_v.at[slot, 1],
                         si[p])

    def wait_idx(p):
        for _ in range(2):
            pltpu.make_async_copy(src_hbm.at[pl.ds(0, CHUNK)],
                                  idx_v.at[0, 0], si[p]).wait()

    def issue_xe(k, bx, be_):
        slot = lax.rem(k, NSLOT)
        pltpu.async_copy(x_hbm.at[idx_v.at[slot, 0]], xb[bx], sx[bx])
        pltpu.async_copy(emb_hbm.at[pl.ds((start + k) * CHUNK, CHUNK)],
                         eb[be_], se[be_])

    def wait_xe(bx, be_):
        pltpu.make_async_copy(emb_hbm.at[pl.ds(0, CHUNK)], xb[bx],
                              sx[bx]).wait()
        pltpu.make_async_copy(emb_hbm.at[pl.ds(0, CHUNK)], eb[be_],
                              se[be_]).wait()

    def compute(bx, be_):
        @pl.loop(0, CHUNK)
        def _c(r):
            for j in range(D // 16):
                sl = (pl.ds(r, 1), pl.ds(j * 16, 16))
                eb[be_].at[sl][...] = jnp.maximum(
                    xb[bx].at[sl][...] + eb[be_].at[sl][...], 0.0)

    def issue_scatter(k, be_):
        pltpu.async_copy(eb[be_], aggr_sh.at[idx_v.at[lax.rem(k, NSLOT), 1]],
                         ss[be_], add=True)

    def wait_scatter(be_):
        pltpu.make_async_copy(eb[0], aggr_sh.at[pl.ds(0, CHUNK)],
                              ss[be_]).wait()

    def body(i, bx, be_, do_d=True, do_e=True, do_f=True):
        wait_xe(bx, be_)
        compute(bx, be_)
        issue_scatter(i, be_)
        if do_d:
            wait_scatter((be_ + 2) % 3)
        if do_e:
            wait_idx(bx)
            issue_xe(i + 2, bx, (be_ + 2) % 3)
        if do_f:
            issue_idx(i + 3, 1 - bx)

    issue_idx(0, 0)
    issue_idx(1, 1)
    wait_idx(0)
    issue_xe(0, 0, 0)
    issue_idx(2, 0)
    wait_idx(1)
    issue_xe(1, 1, 1)

    body(0, 0, 0, do_d=False)
    body(1, 1, 1)

    @pl.loop(0, (BASE_CH - 6) // 6)
    def _main(g):
        i0 = 2 + 6 * g
        body(i0 + 0, 0, 2)
        body(i0 + 1, 1, 0)
        body(i0 + 2, 0, 1)
        body(i0 + 3, 1, 2)
        body(i0 + 4, 0, 0)
        body(i0 + 5, 1, 1)

    body(BASE_CH - 4, 0, 2)

    def tail_153():
        body(BASE_CH - 3, 1, 0, do_f=False)

        @pl.when(wid < EXTRA)
        def _f():
            issue_idx(BASE_CH, 0)

    tail_153()

    def tail_154():
        body(BASE_CH - 2, 0, 1, do_e=False, do_f=False)

        @pl.when(wid < EXTRA)
        def _e():
            wait_idx(0)
            issue_xe(BASE_CH, 0, 0)

    tail_154()
    body(BASE_CH - 1, 1, 2, do_e=False, do_f=False)

    @pl.when(wid < EXTRA)
    def _extra():
        body(BASE_CH, 0, 0, do_e=False, do_f=False)

    @pl.when(wid < EXTRA)
    def _d0():
        wait_scatter(0)

    @pl.when(wid >= EXTRA)
    def _d2():
        wait_scatter(2)

    plsc.subcore_barrier()

    def flush(r, n):
        pltpu.sync_copy(aggr_sh.at[pl.ds(r, n)], xb0.at[pl.ds(0, n)])
        pltpu.sync_copy(xb0.at[pl.ds(0, n)], out_hbm.at[c, pl.ds(r, n)])

    for k in range(ROWS0 // CHUNK):
        flush(row0 + k * CHUNK, CHUNK)
    flush(row0 + 576, 48)

    @pl.when(s == NS - 1)
    def _w1():
        flush(NS * ROWS0, 16)


BN = 1000


def _mlp_body(x_ref, a0_ref, a1_ref, eps_ref, w1_ref, b1_ref, g1_ref, bt1_ref,
              m1_ref, v1_ref, w2_ref, b2_ref, g2_ref, bt2_ref, m2_ref, v2_ref,
              o_ref):
    h = x_ref[...] * (1.0 + eps_ref[0, 0]) + a0_ref[0] + a1_ref[0]
    t = jnp.dot(h, w1_ref[...], preferred_element_type=jnp.float32) + b1_ref[...]
    t = (t - m1_ref[...]) * lax.rsqrt(v1_ref[...] + 1e-5) * g1_ref[...] + bt1_ref[...]
    t = jnp.maximum(t, 0.0)
    o = jnp.dot(t, w2_ref[...], preferred_element_type=jnp.float32) + b2_ref[...]
    o_ref[...] = (o - m2_ref[...]) * lax.rsqrt(v2_ref[...] + 1e-5) * g2_ref[...] + bt2_ref[...]


def _mlp(x, aggr, eps2, W1, b1, g1, bt1, m1, v1, W2, b2, g2, bt2, m2, v2):
    row = lambda i: (i, 0)
    const2 = lambda i: (0, 0)
    return pl.pallas_call(
        _mlp_body,
        grid=(N // BN,),
        in_specs=[
            pl.BlockSpec((BN, D), row),
            pl.BlockSpec((1, BN, D), lambda i: (0, i, 0)),
            pl.BlockSpec((1, BN, D), lambda i: (1, i, 0)),
            pl.BlockSpec((1, 1), const2),
            pl.BlockSpec((D, DH), const2),
            pl.BlockSpec((1, DH), const2),
            pl.BlockSpec((1, DH), const2),
            pl.BlockSpec((1, DH), const2),
            pl.BlockSpec((1, DH), const2),
            pl.BlockSpec((1, DH), const2),
            pl.BlockSpec((DH, D), const2),
            pl.BlockSpec((1, D), const2),
            pl.BlockSpec((1, D), const2),
            pl.BlockSpec((1, D), const2),
            pl.BlockSpec((1, D), const2),
            pl.BlockSpec((1, D), const2),
        ],
        out_specs=pl.BlockSpec((BN, D), row),
        out_shape=jax.ShapeDtypeStruct((N, D), jnp.float32),
    )(x, aggr, aggr, eps2, W1, b1, g1, bt1, m1, v1, W2, b2, g2, bt2, m2, v2)


def kernel(input_feature, edge_index, edge_attr, W_e, b_e, eps, W1, b1,
           gamma1, beta1, mean1, var1, W2, b2, gamma2, beta2, mean2, var2):
    src = edge_index[0]
    dst = edge_index[1]
    emb = _edge_emb(edge_attr, W_e, b_e.reshape(1, D))
    aggr = _sc_aggr(emb, src, dst, input_feature)
    out = _mlp(
        input_feature, aggr, jnp.reshape(eps, (1, 1)),
        W1, b1.reshape(1, DH), gamma1.reshape(1, DH), beta1.reshape(1, DH),
        mean1.reshape(1, DH), var1.reshape(1, DH),
        W2, b2.reshape(1, D), gamma2.reshape(1, D), beta2.reshape(1, D),
        mean2.reshape(1, D), var2.reshape(1, D),
    )
    return out


# speedup vs baseline: 5.7711x; 1.3046x over previous
import functools
import jax
import jax.numpy as jnp
from jax import lax
from jax.experimental import pallas as pl
from jax.experimental.pallas import tpu as pltpu
from jax.experimental.pallas import tpu_sc as plsc

N = 10000
E = 320000
D = 128
DE = 16
DH = 256

NC = 2
NS = 16
NW = NC * NS
CHUNK = 64
NCHUNK = E // CHUNK
BASE_CH = NCHUNK // NW
EXTRA = NCHUNK - BASE_CH * NW
ROWS0 = 624
NSLOT = 6


BE = 3200


def _emb_body(attr_t_ref, we_ref, be_ref, out_ref):
    out_ref[...] = lax.dot_general(
        attr_t_ref[...], we_ref[...],
        (((0,), (0,)), ((), ())),
        preferred_element_type=jnp.float32,
    ) + be_ref[...]


def _edge_emb(edge_attr_t, W_e, b_e2):
    return pl.pallas_call(
        _emb_body,
        grid=(E // BE,),
        in_specs=[
            pl.BlockSpec((DE, BE), lambda i: (0, i)),
            pl.BlockSpec((DE, D), lambda i: (0, 0)),
            pl.BlockSpec((1, D), lambda i: (0, 0)),
        ],
        out_specs=pl.BlockSpec((BE, D), lambda i: (i, 0)),
        out_shape=jax.ShapeDtypeStruct((E, D), jnp.float32),
    )(edge_attr_t, W_e, b_e2)


_sc_mesh = plsc.VectorSubcoreMesh(
    core_axis_name="core", subcore_axis_name="subcore",
    num_cores=NC, num_subcores=NS,
)


@functools.partial(
    pl.kernel,
    mesh=_sc_mesh,
    out_type=jax.ShapeDtypeStruct((NC, N, D), jnp.float32),
    scratch_types=[
        pltpu.VMEM((NSLOT, 2, CHUNK), jnp.int32),
        pltpu.VMEM((CHUNK, D), jnp.float32),
        pltpu.VMEM((CHUNK, D), jnp.float32),
        pltpu.VMEM((CHUNK, D), jnp.float32),
        pltpu.VMEM((CHUNK, D), jnp.float32),
        pltpu.VMEM((CHUNK, D), jnp.float32),
        pltpu.VMEM_SHARED((N, D), jnp.float32),
        pltpu.SemaphoreType.DMA,
        pltpu.SemaphoreType.DMA,
        pltpu.SemaphoreType.DMA,
        pltpu.SemaphoreType.DMA,
        pltpu.SemaphoreType.DMA,
        pltpu.SemaphoreType.DMA,
        pltpu.SemaphoreType.DMA,
        pltpu.SemaphoreType.DMA,
        pltpu.SemaphoreType.DMA,
        pltpu.SemaphoreType.DMA,
    ],
)
def _sc_aggr(emb_hbm, src_hbm, dst_hbm, x_hbm, out_hbm,
             idx_v, xb0, xb1, eb0, eb1, eb2, aggr_sh,
             si0, si1, sx0, sx1, se0, se1, se2, ss0, ss1, ss2):
    c = lax.axis_index("core")
    s = lax.axis_index("subcore")
    wid = c * NS + s
    xb = (xb0, xb1)
    eb = (eb0, eb1, eb2)
    si = (si0, si1)
    sx = (sx0, sx1)
    se = (se0, se1, se2)
    ss = (ss0, ss1, ss2)

    @pl.loop(0, CHUNK)
    def _z(r):
        for j in range(D // 16):
            xb0.at[pl.ds(r, 1), pl.ds(j * 16, 16)][...] = jnp.zeros(
                (1, 16), jnp.float32)

    row0 = s * ROWS0
    for k in range(ROWS0 // CHUNK):
        pltpu.sync_copy(xb0, aggr_sh.at[pl.ds(row0 + k * CHUNK, CHUNK)])
    pltpu.sync_copy(xb0.at[pl.ds(0, 48)],
                    aggr_sh.at[pl.ds(row0 + 576, 48)])

    @pl.when(s == NS - 1)
    def _z1():
        pltpu.sync_copy(xb0.at[pl.ds(0, 16)], aggr_sh.at[pl.ds(NS * ROWS0, 16)])

    plsc.subcore_barrier()

    start = wid * BASE_CH + jnp.minimum(wid, EXTRA)

    def issue_idx(k, p):
        base = (start + k) * CHUNK
        slot = lax.rem(k, NSLOT)
        pltpu.async_copy(src_hbm.at[pl.ds(base, CHUNK)], idx_v.at[slot, 0],
                         si[p])
        pltpu.async_copy(dst_hbm.at[pl.ds(base, CHUNK)], idx_v.at[slot, 1],
                         si[p])

    def wait_idx(p):
        for _ in range(2):
            pltpu.make_async_copy(src_hbm.at[pl.ds(0, CHUNK)],
                                  idx_v.at[0, 0], si[p]).wait()

    def issue_xe(k, bx, be_):
        slot = lax.rem(k, NSLOT)
        pltpu.async_copy(x_hbm.at[idx_v.at[slot, 0]], xb[bx], sx[bx])
        pltpu.async_copy(emb_hbm.at[pl.ds((start + k) * CHUNK, CHUNK)],
                         eb[be_], se[be_])

    def wait_xe(bx, be_):
        pltpu.make_async_copy(emb_hbm.at[pl.ds(0, CHUNK)], xb[bx],
                              sx[bx]).wait()
        pltpu.make_async_copy(emb_hbm.at[pl.ds(0, CHUNK)], eb[be_],
                              se[be_]).wait()

    def compute(bx, be_):
        @pl.loop(0, CHUNK)
        def _c(r):
            for j in range(D // 16):
                sl = (pl.ds(r, 1), pl.ds(j * 16, 16))
                eb[be_].at[sl][...] = jnp.maximum(
                    xb[bx].at[sl][...] + eb[be_].at[sl][...], 0.0)

    def issue_scatter(k, be_):
        pltpu.async_copy(eb[be_], aggr_sh.at[idx_v.at[lax.rem(k, NSLOT), 1]],
                         ss[be_], add=True)

    def wait_scatter(be_):
        pltpu.make_async_copy(eb[0], aggr_sh.at[pl.ds(0, CHUNK)],
                              ss[be_]).wait()

    def body(i, bx, be_, do_d=True, do_e=True, do_f=True):
        wait_xe(bx, be_)
        compute(bx, be_)
        issue_scatter(i, be_)
        if do_d:
            wait_scatter((be_ + 2) % 3)
        if do_e:
            wait_idx(bx)
            issue_xe(i + 2, bx, (be_ + 2) % 3)
        if do_f:
            issue_idx(i + 3, 1 - bx)

    issue_idx(0, 0)
    issue_idx(1, 1)
    wait_idx(0)
    issue_xe(0, 0, 0)
    issue_idx(2, 0)
    wait_idx(1)
    issue_xe(1, 1, 1)

    body(0, 0, 0, do_d=False)
    body(1, 1, 1)

    @pl.loop(0, (BASE_CH - 6) // 6)
    def _main(g):
        i0 = 2 + 6 * g
        body(i0 + 0, 0, 2)
        body(i0 + 1, 1, 0)
        body(i0 + 2, 0, 1)
        body(i0 + 3, 1, 2)
        body(i0 + 4, 0, 0)
        body(i0 + 5, 1, 1)

    body(BASE_CH - 4, 0, 2)

    def tail_153():
        body(BASE_CH - 3, 1, 0, do_f=False)

        @pl.when(wid < EXTRA)
        def _f():
            issue_idx(BASE_CH, 0)

    tail_153()

    def tail_154():
        body(BASE_CH - 2, 0, 1, do_e=False, do_f=False)

        @pl.when(wid < EXTRA)
        def _e():
            wait_idx(0)
            issue_xe(BASE_CH, 0, 0)

    tail_154()
    body(BASE_CH - 1, 1, 2, do_e=False, do_f=False)

    @pl.when(wid < EXTRA)
    def _extra():
        body(BASE_CH, 0, 0, do_e=False, do_f=False)

    @pl.when(wid < EXTRA)
    def _d0():
        wait_scatter(0)

    @pl.when(wid >= EXTRA)
    def _d2():
        wait_scatter(2)

    plsc.subcore_barrier()

    def flush(r, n):
        pltpu.sync_copy(aggr_sh.at[pl.ds(r, n)], xb0.at[pl.ds(0, n)])
        pltpu.sync_copy(xb0.at[pl.ds(0, n)], out_hbm.at[c, pl.ds(r, n)])

    for k in range(ROWS0 // CHUNK):
        flush(row0 + k * CHUNK, CHUNK)
    flush(row0 + 576, 48)

    @pl.when(s == NS - 1)
    def _w1():
        flush(NS * ROWS0, 16)


BN = 1000


def _mlp_body(x_ref, a0_ref, a1_ref, eps_ref, w1_ref, b1_ref, g1_ref, bt1_ref,
              m1_ref, v1_ref, w2_ref, b2_ref, g2_ref, bt2_ref, m2_ref, v2_ref,
              o_ref):
    h = x_ref[...] * (1.0 + eps_ref[0, 0]) + a0_ref[0] + a1_ref[0]
    t = jnp.dot(h, w1_ref[...], preferred_element_type=jnp.float32) + b1_ref[...]
    t = (t - m1_ref[...]) * lax.rsqrt(v1_ref[...] + 1e-5) * g1_ref[...] + bt1_ref[...]
    t = jnp.maximum(t, 0.0)
    o = jnp.dot(t, w2_ref[...], preferred_element_type=jnp.float32) + b2_ref[...]
    o_ref[...] = (o - m2_ref[...]) * lax.rsqrt(v2_ref[...] + 1e-5) * g2_ref[...] + bt2_ref[...]


def _mlp(x, aggr, eps2, W1, b1, g1, bt1, m1, v1, W2, b2, g2, bt2, m2, v2):
    row = lambda i: (i, 0)
    const2 = lambda i: (0, 0)
    return pl.pallas_call(
        _mlp_body,
        grid=(N // BN,),
        in_specs=[
            pl.BlockSpec((BN, D), row),
            pl.BlockSpec((1, BN, D), lambda i: (0, i, 0)),
            pl.BlockSpec((1, BN, D), lambda i: (1, i, 0)),
            pl.BlockSpec((1, 1), const2),
            pl.BlockSpec((D, DH), const2),
            pl.BlockSpec((1, DH), const2),
            pl.BlockSpec((1, DH), const2),
            pl.BlockSpec((1, DH), const2),
            pl.BlockSpec((1, DH), const2),
            pl.BlockSpec((1, DH), const2),
            pl.BlockSpec((DH, D), const2),
            pl.BlockSpec((1, D), const2),
            pl.BlockSpec((1, D), const2),
            pl.BlockSpec((1, D), const2),
            pl.BlockSpec((1, D), const2),
            pl.BlockSpec((1, D), const2),
        ],
        out_specs=pl.BlockSpec((BN, D), row),
        out_shape=jax.ShapeDtypeStruct((N, D), jnp.float32),
    )(x, aggr, aggr, eps2, W1, b1, g1, bt1, m1, v1, W2, b2, g2, bt2, m2, v2)


def kernel(input_feature, edge_index, edge_attr, W_e, b_e, eps, W1, b1,
           gamma1, beta1, mean1, var1, W2, b2, gamma2, beta2, mean2, var2):
    src = edge_index[0]
    dst = edge_index[1]
    emb = _edge_emb(edge_attr.T, W_e, b_e.reshape(1, D))
    aggr = _sc_aggr(emb, src, dst, input_feature)
    out = _mlp(
        input_feature, aggr, jnp.reshape(eps, (1, 1)),
        W1, b1.reshape(1, DH), gamma1.reshape(1, DH), beta1.reshape(1, DH),
        mean1.reshape(1, DH), var1.reshape(1, DH),
        W2, b2.reshape(1, D), gamma2.reshape(1, D), beta2.reshape(1, D),
        mean2.reshape(1, D), var2.reshape(1, D),
    )
    return out
